# baseline (speedup 1.0000x reference)
import functools

import jax
import jax.numpy as jnp
from jax import lax
from jax.experimental import pallas as pl
from jax.experimental.pallas import tpu as pltpu
from jax.experimental.pallas import tpu_sc as plsc

_NC = 2
_NS = 16
_NW = _NC * _NS
_K = 8


def _make(nb, half):
    bpw = nb // _NW
    nch = bpw // _K
    mesh = plsc.VectorSubcoreMesh(core_axis_name="c", subcore_axis_name="s")

    @functools.partial(
        pl.kernel,
        out_type=jax.ShapeDtypeStruct((nb, 2, half), jnp.float32),
        mesh=mesh,
        scratch_types=[
            pltpu.VMEM((bpw,), jnp.int32),
            pltpu.VMEM((2 * bpw,), jnp.int32),
            pltpu.VMEM((_K, half), jnp.float32),
            pltpu.SemaphoreType.DMA,
        ],
    )
    def emb(idx_hbm, table_hbm, out_hbm, idx_v, idx2_v, buf, gsem):
        wid = lax.axis_index("s") * _NC + lax.axis_index("c")
        base = wid * bpw
        pltpu.sync_copy(idx_hbm.at[pl.ds(base, bpw)], idx_v)
        for v in range(bpw // 16):
            e = idx_v[pl.ds(v * 16, 16)]
            idx2_v[pl.ds(v * 16, 16)] = e * 2
            idx2_v[pl.ds(bpw + v * 16, 16)] = e * 2 + 1
        for h in (0, 1):
            def body(c, carry):
                off = pl.multiple_of(h * bpw + c * _K, 8)
                row = pl.multiple_of(base + c * _K, 8)
                pltpu.async_copy(
                    table_hbm.at[idx2_v.at[pl.ds(off, _K)]], buf, gsem
                ).wait()
                pltpu.sync_copy(buf, out_hbm.at[pl.ds(row, _K), h])
                return carry
            lax.fori_loop(0, nch, body, 0)

    return emb


def kernel(idx, table):
    b, t = idx.shape
    v, d = table.shape
    nb = b * t
    half = d // 2
    idx_flat = idx.reshape(nb).astype(jnp.int32)
    table2 = table.reshape(v * 2, half)
    out = _make(nb, half)(idx_flat, table2)
    return out.reshape(b, t, d)


# speedup vs baseline: 1.0540x; 1.0540x over previous
import functools

import jax
import jax.numpy as jnp
from jax import lax
from jax.experimental import pallas as pl
from jax.experimental.pallas import tpu as pltpu
from jax.experimental.pallas import tpu_sc as plsc

_NC = 2
_NS = 16
_NW = _NC * _NS
_K = 8


def _make(nb, half):
    bpw = nb // _NW
    nch = bpw // _K
    mesh = plsc.VectorSubcoreMesh(core_axis_name="c", subcore_axis_name="s")

    @functools.partial(
        pl.kernel,
        out_type=jax.ShapeDtypeStruct((nb, 2, half), jnp.float32),
        mesh=mesh,
        scratch_types=[
            pltpu.VMEM((bpw,), jnp.int32),
            pltpu.VMEM((2 * bpw,), jnp.int32),
            pltpu.VMEM((_K, half), jnp.float32),
            pltpu.VMEM((_K, half), jnp.float32),
            pltpu.SemaphoreType.DMA,
            pltpu.SemaphoreType.DMA,
            pltpu.SemaphoreType.DMA,
            pltpu.SemaphoreType.DMA,
        ],
    )
    def emb(idx_hbm, table_hbm, out_hbm, idx_v, idx2_v, buf0, buf1,
            g0, g1, w0, w1):
        wid = lax.axis_index("s") * _NC + lax.axis_index("c")
        base = wid * bpw
        pltpu.sync_copy(idx_hbm.at[pl.ds(base, bpw)], idx_v)
        for v in range(bpw // 16):
            e = idx_v[pl.ds(v * 16, 16)] * 2
            idx2_v[pl.ds(v * 16, 16)] = e
            idx2_v[pl.ds(bpw + v * 16, 16)] = e + 1

        def start_g(h, c, buf, sem):
            off = pl.multiple_of(h * bpw + c * _K, 8)
            return pltpu.async_copy(
                table_hbm.at[idx2_v.at[pl.ds(off, _K)]], buf, sem)

        def start_w(h, c, buf, sem):
            row = pl.multiple_of(base + c * _K, 8)
            return pltpu.async_copy(buf, out_hbm.at[pl.ds(row, _K), h], sem)

        def wait_g(buf, sem):
            pltpu.make_async_copy(table_hbm.at[pl.ds(0, _K)], buf, sem).wait()

        def wait_w(h, buf, sem):
            pltpu.make_async_copy(buf, out_hbm.at[pl.ds(0, _K), h], sem).wait()

        for h in (0, 1):
            start_g(h, 0, buf0, g0)
            start_g(h, 1, buf1, g1)

            def body(i, carry):
                c = 2 * i
                wait_g(buf0, g0)
                start_w(h, c, buf0, w0)
                wait_g(buf1, g1)
                start_w(h, c + 1, buf1, w1)
                wait_w(h, buf0, w0)
                start_g(h, c + 2, buf0, g0)
                wait_w(h, buf1, w1)
                start_g(h, c + 3, buf1, g1)
                return carry

            lax.fori_loop(0, nch // 2 - 1, body, 0)
            wait_g(buf0, g0)
            start_w(h, nch - 2, buf0, w0)
            wait_g(buf1, g1)
            start_w(h, nch - 1, buf1, w1)
            wait_w(h, buf0, w0)
            wait_w(h, buf1, w1)

    return emb


def kernel(idx, table):
    b, t = idx.shape
    v, d = table.shape
    nb = b * t
    half = d // 2
    idx_flat = idx.reshape(nb).astype(jnp.int32)
    table2 = table.reshape(v * 2, half)
    out = _make(nb, half)(idx_flat, table2)
    return out.reshape(b, t, d)


# speedup vs baseline: 4.0303x; 3.8238x over previous
import functools

import jax
import jax.numpy as jnp
from jax import lax
from jax.experimental import pallas as pl
from jax.experimental.pallas import tpu as pltpu
from jax.experimental.pallas import tpu_sc as plsc

_NC = 2
_NS = 16
_NW = _NC * _NS
_K = 8


def _make(nb, d):
    half = d // 2
    bpw = nb // _NW
    nch = bpw // _K
    mesh = plsc.VectorSubcoreMesh(core_axis_name="c", subcore_axis_name="s")

    @functools.partial(
        pl.kernel,
        out_type=jax.ShapeDtypeStruct((nb, d), jnp.float32),
        mesh=mesh,
        scratch_types=[
            pltpu.VMEM((bpw,), jnp.int32),
            pltpu.VMEM((_K, half), jnp.float32),
            pltpu.VMEM((_K, half), jnp.float32),
            pltpu.SemaphoreType.DMA,
            pltpu.SemaphoreType.DMA,
            pltpu.SemaphoreType.DMA,
            pltpu.SemaphoreType.DMA,
        ],
    )
    def emb(idx_hbm, table_hbm, out_hbm, idx_v, buf0, buf1, g0, g1, w0, w1):
        wid = lax.axis_index("s") * _NC + lax.axis_index("c")
        base = wid * bpw
        pltpu.sync_copy(idx_hbm.at[pl.ds(base, bpw)], idx_v)

        def start_g(h, c, buf, sem):
            off = pl.multiple_of(c * _K, 8)
            return pltpu.async_copy(
                table_hbm.at[idx_v.at[pl.ds(off, _K)], pl.ds(h * half, half)],
                buf, sem)

        def start_w(h, c, buf, sem):
            row = pl.multiple_of(base + c * _K, 8)
            return pltpu.async_copy(
                buf, out_hbm.at[pl.ds(row, _K), pl.ds(h * half, half)], sem)

        def wait_g(buf, sem):
            pltpu.make_async_copy(
                table_hbm.at[pl.ds(0, _K), pl.ds(0, half)], buf, sem).wait()

        def wait_w(buf, sem):
            pltpu.make_async_copy(
                buf, out_hbm.at[pl.ds(0, _K), pl.ds(0, half)], sem).wait()

        for h in (0, 1):
            start_g(h, 0, buf0, g0)
            start_g(h, 1, buf1, g1)

            def body(i, carry):
                c = 2 * i
                wait_g(buf0, g0)
                start_w(h, c, buf0, w0)
                wait_g(buf1, g1)
                start_w(h, c + 1, buf1, w1)
                wait_w(buf0, w0)
                start_g(h, c + 2, buf0, g0)
                wait_w(buf1, w1)
                start_g(h, c + 3, buf1, g1)
                return carry

            lax.fori_loop(0, nch // 2 - 1, body, 0)
            wait_g(buf0, g0)
            start_w(h, nch - 2, buf0, w0)
            wait_g(buf1, g1)
            start_w(h, nch - 1, buf1, w1)
            wait_w(buf0, w0)
            wait_w(buf1, w1)

    return emb


def kernel(idx, table):
    b, t = idx.shape
    v, d = table.shape
    nb = b * t
    idx_flat = idx.reshape(nb).astype(jnp.int32)
    out = _make(nb, d)(idx_flat, table)
    return out.reshape(b, t, d)


# speedup vs baseline: 4.1923x; 1.0402x over previous
import functools

import jax
import jax.numpy as jnp
from jax import lax
from jax.experimental import pallas as pl
from jax.experimental.pallas import tpu as pltpu
from jax.experimental.pallas import tpu_sc as plsc

_NC = 2
_NS = 16
_NW = _NC * _NS
_K = 8
_NBUF = 3


def _make(nb, d):
    half = d // 2
    bpw = nb // _NW
    nch = bpw // _K
    nst = 2 * nch
    mesh = plsc.VectorSubcoreMesh(core_axis_name="c", subcore_axis_name="s")

    @functools.partial(
        pl.kernel,
        out_type=jax.ShapeDtypeStruct((nb, d), jnp.float32),
        mesh=mesh,
        scratch_types=(
            [pltpu.VMEM((bpw,), jnp.int32)]
            + [pltpu.VMEM((_K, half), jnp.float32)] * _NBUF
            + [pltpu.SemaphoreType.DMA] * (2 * _NBUF)
        ),
    )
    def emb(idx_hbm, table_hbm, out_hbm, idx_v, *rest):
        bufs = rest[:_NBUF]
        gsem = rest[_NBUF:2 * _NBUF]
        wsem = rest[2 * _NBUF:]
        wid = lax.axis_index("s") * _NC + lax.axis_index("c")
        base = wid * bpw
        pltpu.sync_copy(idx_hbm.at[pl.ds(base, bpw)], idx_v)

        def coords(s):
            scaled = jnp.int32(s)
            col = pl.multiple_of((scaled // nch) * half, half)
            off = pl.multiple_of((scaled % nch) * _K, 8)
            return off, col

        def start_g(s, buf, sem):
            off, col = coords(s)
            return pltpu.async_copy(
                table_hbm.at[idx_v.at[pl.ds(off, _K)], pl.ds(col, half)],
                buf, sem)

        def start_w(s, buf, sem):
            off, col = coords(s)
            row = pl.multiple_of(base, 8) + off
            return pltpu.async_copy(
                buf, out_hbm.at[pl.ds(row, _K), pl.ds(col, half)], sem)

        def wait_g(buf, sem):
            pltpu.make_async_copy(
                table_hbm.at[pl.ds(0, _K), pl.ds(0, half)], buf, sem).wait()

        def wait_w(buf, sem):
            pltpu.make_async_copy(
                buf, out_hbm.at[pl.ds(0, _K), pl.ds(0, half)], sem).wait()

        for b in range(_NBUF):
            start_g(b, bufs[b], gsem[b])

        nloop = nst // _NBUF - 1
        def body(k, carry):
            s0 = k * _NBUF
            for b in range(_NBUF):
                wait_g(bufs[b], gsem[b])
                start_w(s0 + b, bufs[b], wsem[b])
                wait_w(bufs[b], wsem[b])
                start_g(s0 + b + _NBUF, bufs[b], gsem[b])
            return carry

        lax.fori_loop(0, nloop, body, 0)
        s0 = nloop * _NBUF
        rem = nst - s0
        for j in range(rem):
            b = (s0 + j) % _NBUF
            wait_g(bufs[b], gsem[b])
            start_w(s0 + j, bufs[b], wsem[b])
            if s0 + j + _NBUF < nst:
                wait_w(bufs[b], wsem[b])
                start_g(s0 + j + _NBUF, bufs[b], gsem[b])
        for j in range(min(rem, _NBUF)):
            b = (nst - 1 - j) % _NBUF
            wait_w(bufs[b], wsem[b])

    return emb


def kernel(idx, table):
    b, t = idx.shape
    v, d = table.shape
    nb = b * t
    idx_flat = idx.reshape(nb).astype(jnp.int32)
    out = _make(nb, d)(idx_flat, table)
    return out.reshape(b, t, d)
